# Initial kernel scaffold; baseline (speedup 1.0000x reference)
import jax
import jax.numpy as jnp
from jax.experimental import pallas as pl


def kernel(x, hs, key_w, key_b, hs_value_w, query_w):
    raise NotImplementedError("write your pallas kernel here")


# speedup vs baseline: 4.5412x; 4.5412x over previous
import math

import jax
import jax.numpy as jnp
from jax.experimental import pallas as pl

IN = 64
HID = 64
NU = 128
TOPK = 8
KS = 64
QS = 64
VS = 64
RB = 256


def _fused_body(x_ref, hs_ref, key_w_ref, qw_ref, w3_ref, out_ref):
    xb = x_ref[...]
    q = jnp.sum(hs_ref[...][:, :, None] * qw_ref[...], axis=1)
    k = jax.lax.dot_general(xb, key_w_ref[...], (((1,), (0,)), ((), ())),
                            preferred_element_type=jnp.float32)
    s = jax.lax.dot_general(k, q, (((1,), (1,)), ((), ())),
                            preferred_element_type=jnp.float32)
    s = s * (1.0 / math.sqrt(KS))

    iota = jax.lax.broadcasted_iota(jnp.int32, (RB, NU), 1)
    scur = s
    selected = jnp.zeros((RB, NU), dtype=jnp.bool_)
    for _ in range(TOPK):
        m = jnp.max(scur, axis=1, keepdims=True)
        ismax = scur == m
        first = jnp.min(jnp.where(ismax, iota, NU), axis=1, keepdims=True)
        sel = iota == first
        selected = jnp.logical_or(selected, sel)
        scur = jnp.where(sel, -jnp.inf, scur)

    w = jnp.where(selected, 1.0 / (1.0 + jnp.exp(-s)), 0.0)

    v = jax.lax.dot_general(xb, w3_ref[...], (((1,), (0,)), ((), ())),
                            preferred_element_type=jnp.float32)
    v3 = v.reshape(RB, VS, NU)
    out_ref[...] = jnp.sum(v3 * w[:, None, :], axis=2)


def kernel(x, hs, key_w, key_b, hs_value_w, query_w):
    del key_b
    b = x.shape[0]
    x2 = x.reshape(b, IN)
    w3 = jnp.transpose(hs_value_w, (1, 2, 0)).reshape(IN, VS * NU)
    out = pl.pallas_call(
        _fused_body,
        grid=(b // RB,),
        in_specs=[
            pl.BlockSpec((RB, IN), lambda i: (i, 0)),
            pl.BlockSpec((NU, HID), lambda i: (0, 0)),
            pl.BlockSpec((IN, KS), lambda i: (0, 0)),
            pl.BlockSpec((NU, HID, QS), lambda i: (0, 0, 0)),
            pl.BlockSpec((IN, VS * NU), lambda i: (0, 0)),
        ],
        out_specs=pl.BlockSpec((RB, VS), lambda i: (i, 0)),
        out_shape=jax.ShapeDtypeStruct((b, VS), jnp.float32),
    )(x2, hs, key_w, query_w, w3)
    return out


# speedup vs baseline: 5.4974x; 1.2106x over previous
import math

import jax
import jax.numpy as jnp
from jax.experimental import pallas as pl
from jax.experimental.pallas import tpu as pltpu

IN = 64
HID = 64
NU = 128
TOPK = 8
KS = 64
QS = 64
VS = 64
RB = 256


def _fused_body(x_ref, hs_ref, key_w_ref, qw_ref, w3_ref, e_ref, out_ref,
                q_ref):
    @pl.when(pl.program_id(0) == 0)
    def _():
        q_ref[...] = jnp.sum(hs_ref[...][:, :, None] * qw_ref[...], axis=1)

    xb = x_ref[...]
    q = q_ref[...]
    k = jax.lax.dot_general(xb, key_w_ref[...], (((1,), (0,)), ((), ())),
                            preferred_element_type=jnp.float32)
    s = jax.lax.dot_general(k, q, (((1,), (1,)), ((), ())),
                            preferred_element_type=jnp.float32)
    s = s * (1.0 / math.sqrt(KS))

    iota = jax.lax.broadcasted_iota(jnp.int32, (RB, NU), 1)
    scur = s
    selected = jnp.zeros((RB, NU), dtype=jnp.bool_)
    for _ in range(TOPK):
        m = jnp.max(scur, axis=1, keepdims=True)
        ismax = scur == m
        first = jnp.min(jnp.where(ismax, iota, NU), axis=1, keepdims=True)
        sel = iota == first
        selected = jnp.logical_or(selected, sel)
        scur = jnp.where(sel, -jnp.inf, scur)

    w = jnp.where(selected, 1.0 / (1.0 + jnp.exp(-s)), 0.0)

    v = jax.lax.dot_general(xb, w3_ref[...], (((1,), (0,)), ((), ())),
                            preferred_element_type=jnp.float32)
    p = v * jnp.tile(w, (1, VS))
    out_ref[...] = jax.lax.dot_general(p, e_ref[...], (((1,), (0,)), ((), ())),
                                       preferred_element_type=jnp.float32)


def kernel(x, hs, key_w, key_b, hs_value_w, query_w):
    del key_b
    b = x.shape[0]
    x2 = x.reshape(b, IN)
    w3 = jnp.transpose(hs_value_w, (1, 2, 0)).reshape(IN, VS * NU)
    e = (jnp.arange(VS * NU, dtype=jnp.int32)[:, None] // NU
         == jnp.arange(VS, dtype=jnp.int32)[None, :]).astype(jnp.float32)
    out = pl.pallas_call(
        _fused_body,
        grid=(b // RB,),
        in_specs=[
            pl.BlockSpec((RB, IN), lambda i: (i, 0)),
            pl.BlockSpec((NU, HID), lambda i: (0, 0)),
            pl.BlockSpec((IN, KS), lambda i: (0, 0)),
            pl.BlockSpec((NU, HID, QS), lambda i: (0, 0, 0)),
            pl.BlockSpec((IN, VS * NU), lambda i: (0, 0)),
            pl.BlockSpec((VS * NU, VS), lambda i: (0, 0)),
        ],
        out_specs=pl.BlockSpec((RB, VS), lambda i: (i, 0)),
        out_shape=jax.ShapeDtypeStruct((b, VS), jnp.float32),
        scratch_shapes=[pltpu.VMEM((NU, QS), jnp.float32)],
    )(x2, hs, key_w, query_w, w3, e)
    return out


# speedup vs baseline: 7.1821x; 1.3065x over previous
import math

import jax
import jax.numpy as jnp
import numpy as np
from jax.experimental import pallas as pl
from jax.experimental.pallas import tpu as pltpu

IN = 64
HID = 64
NU = 128
TOPK = 8
KS = 64
QS = 64
VS = 64
RB = 512


def _fused_body(x_ref, hs_ref, key_w_ref, qw_ref, w3_ref, e_ref, out_ref,
                q_ref):
    @pl.when(pl.program_id(0) == 0)
    def _():
        q_ref[...] = jnp.sum(hs_ref[...][:, :, None] * qw_ref[...], axis=1)

    xb = x_ref[...]
    q = q_ref[...]
    k = jax.lax.dot_general(xb, key_w_ref[...], (((1,), (0,)), ((), ())),
                            preferred_element_type=jnp.float32)
    s = jax.lax.dot_general(k, q, (((1,), (1,)), ((), ())),
                            preferred_element_type=jnp.float32)
    s = s * (1.0 / math.sqrt(KS))

    iota = jax.lax.broadcasted_iota(jnp.int32, (RB, NU), 1)
    scur = s
    selected = jnp.zeros((RB, NU), dtype=jnp.bool_)
    for _ in range(TOPK):
        m = jnp.max(scur, axis=1, keepdims=True)
        ismax = scur == m
        first = jnp.min(jnp.where(ismax, iota, NU), axis=1, keepdims=True)
        sel = iota == first
        selected = jnp.logical_or(selected, sel)
        scur = jnp.where(sel, -jnp.inf, scur)

    w = jnp.where(selected, 1.0 / (1.0 + jnp.exp(-s)), 0.0)

    v = jax.lax.dot_general(xb, w3_ref[...], (((1,), (0,)), ((), ())),
                            preferred_element_type=jnp.float32)
    p = v * jnp.tile(w, (1, VS))
    out_ref[...] = jax.lax.dot_general(p, e_ref[...], (((1,), (0,)), ((), ())),
                                       preferred_element_type=jnp.float32)


def kernel(x, hs, key_w, key_b, hs_value_w, query_w):
    del key_b
    b = x.shape[0]
    x2 = x.reshape(b, IN)
    w3 = jnp.transpose(hs_value_w, (1, 2, 0)).reshape(IN, VS * NU)
    e = jnp.asarray((np.arange(VS * NU)[:, None] // NU
                     == np.arange(VS)[None, :]).astype(np.float32))
    out = pl.pallas_call(
        _fused_body,
        grid=(b // RB,),
        in_specs=[
            pl.BlockSpec((RB, IN), lambda i: (i, 0)),
            pl.BlockSpec((NU, HID), lambda i: (0, 0)),
            pl.BlockSpec((IN, KS), lambda i: (0, 0)),
            pl.BlockSpec((NU, HID, QS), lambda i: (0, 0, 0)),
            pl.BlockSpec((IN, VS * NU), lambda i: (0, 0)),
            pl.BlockSpec((VS * NU, VS), lambda i: (0, 0)),
        ],
        out_specs=pl.BlockSpec((RB, VS), lambda i: (i, 0)),
        out_shape=jax.ShapeDtypeStruct((b, VS), jnp.float32),
        scratch_shapes=[pltpu.VMEM((NU, QS), jnp.float32)],
    )(x2, hs, key_w, query_w, w3, e)
    return out


# speedup vs baseline: 7.1854x; 1.0005x over previous
import math

import jax
import jax.numpy as jnp
import numpy as np
from jax.experimental import pallas as pl
from jax.experimental.pallas import tpu as pltpu

IN = 64
HID = 64
NU = 128
TOPK = 8
KS = 64
QS = 64
VS = 64
RB = 512


def _fused_body(x_ref, hs_ref, key_w_ref, qw_ref, w3_ref, e_ref, out_ref,
                q_ref):
    @pl.when(pl.program_id(0) == 0)
    def _():
        q_ref[...] = jnp.sum(hs_ref[...][:, :, None] * qw_ref[...], axis=1)

    xb = x_ref[...]
    q = q_ref[...]
    k = jax.lax.dot_general(xb, key_w_ref[...], (((1,), (0,)), ((), ())),
                            preferred_element_type=jnp.float32)
    s = jax.lax.dot_general(k, q, (((1,), (1,)), ((), ())),
                            preferred_element_type=jnp.float32)
    s = s * (1.0 / math.sqrt(KS))

    iota = jax.lax.broadcasted_iota(jnp.int32, (RB, NU), 1)
    scur = s
    selected = jnp.zeros((RB, NU), dtype=jnp.bool_)
    for _ in range(TOPK):
        m = jnp.max(scur, axis=1, keepdims=True)
        ismax = scur == m
        first = jnp.min(jnp.where(ismax, iota, NU), axis=1, keepdims=True)
        sel = iota == first
        selected = jnp.logical_or(selected, sel)
        scur = jnp.where(sel, -jnp.inf, scur)

    w = jnp.where(selected, 1.0 / (1.0 + jnp.exp(-s)), 0.0)

    v = jax.lax.dot_general(xb.astype(jnp.bfloat16), w3_ref[...],
                            (((1,), (0,)), ((), ())),
                            preferred_element_type=jnp.float32)
    p = v * jnp.tile(w, (1, VS))
    out_ref[...] = jax.lax.dot_general(p, e_ref[...], (((1,), (0,)), ((), ())),
                                       preferred_element_type=jnp.float32)


def kernel(x, hs, key_w, key_b, hs_value_w, query_w):
    del key_b
    b = x.shape[0]
    x2 = x.reshape(b, IN)
    w3 = jnp.transpose(hs_value_w, (1, 2, 0)).reshape(IN, VS * NU)
    w3 = w3.astype(jnp.bfloat16)
    e = jnp.asarray((np.arange(VS * NU)[:, None] // NU
                     == np.arange(VS)[None, :]).astype(np.float32))
    out = pl.pallas_call(
        _fused_body,
        grid=(b // RB,),
        in_specs=[
            pl.BlockSpec((RB, IN), lambda i: (i, 0)),
            pl.BlockSpec((NU, HID), lambda i: (0, 0)),
            pl.BlockSpec((IN, KS), lambda i: (0, 0)),
            pl.BlockSpec((NU, HID, QS), lambda i: (0, 0, 0)),
            pl.BlockSpec((IN, VS * NU), lambda i: (0, 0)),
            pl.BlockSpec((VS * NU, VS), lambda i: (0, 0)),
        ],
        out_specs=pl.BlockSpec((RB, VS), lambda i: (i, 0)),
        out_shape=jax.ShapeDtypeStruct((b, VS), jnp.float32),
        scratch_shapes=[pltpu.VMEM((NU, QS), jnp.float32)],
    )(x2, hs, key_w, query_w, w3, e)
    return out
